# baseline (speedup 1.0000x reference)
import functools

import jax
import jax.numpy as jnp
from jax import lax
from jax.experimental import pallas as pl
from jax.experimental.pallas import tpu as pltpu
from jax.experimental.pallas import tpu_sc as plsc

_BT = 256


def _encode_argmin_body(a_ref, w0_ref, b0_ref, w1_ref, b1_ref, w2_ref,
                        b2_ref, cb_ref, cbn_ref, lat_ref, ind_ref):
    h = jnp.maximum(a_ref[...] @ w0_ref[...] + b0_ref[...], 0.0)
    h = jnp.maximum(h @ w1_ref[...] + b1_ref[...], 0.0)
    lat = h @ w2_ref[...] + b2_ref[...]
    lat_ref[...] = lat
    fn = jnp.sum(lat ** 2, axis=1, keepdims=True)
    dots = lax.dot_general(lat, cb_ref[...], (((1,), (1,)), ((), ())))
    dist = (fn + cbn_ref[...]) - 2.0 * dots
    m = jnp.min(dist, axis=1, keepdims=True)
    k = dist.shape[1]
    iota = lax.broadcasted_iota(jnp.int32, dist.shape, 1)
    ind = jnp.min(jnp.where(dist == m, iota, k), axis=1)
    ind_ref[...] = ind.reshape(1, 1, _BT)


def _encode_argmin(action, enc_w0, enc_b0, enc_w1, enc_b1, enc_w2, enc_b2,
                   codebook, cbn):
    b, a = action.shape
    k, d = codebook.shape
    h = enc_w0.shape[1]
    nb = b // _BT
    full = lambda shape: pl.BlockSpec(shape, lambda i: (0,) * len(shape))
    return pl.pallas_call(
        _encode_argmin_body,
        grid=(nb,),
        in_specs=[
            pl.BlockSpec((_BT, a), lambda i: (i, 0)),
            full((a, h)), full((1, h)),
            full((h, h)), full((1, h)),
            full((h, d)), full((1, d)),
            full((k, d)), full((1, k)),
        ],
        out_specs=[
            pl.BlockSpec((_BT, d), lambda i: (i, 0)),
            pl.BlockSpec((1, 1, _BT), lambda i: (i, 0, 0)),
        ],
        out_shape=[
            jax.ShapeDtypeStruct((b, d), jnp.float32),
            jax.ShapeDtypeStruct((nb, 1, _BT), jnp.int32),
        ],
    )(action, enc_w0, enc_b0, enc_w1, enc_b1, enc_w2, enc_b2, codebook, cbn)


def _sc_gather(codebook_padded, inds):
    b = inds.shape[0]
    k, d = codebook_padded.shape
    nc, ns = 2, 16
    nw = nc * ns
    b_per_w = b // nw
    mesh = plsc.VectorSubcoreMesh(core_axis_name="c", subcore_axis_name="s")

    @functools.partial(
        pl.kernel, mesh=mesh,
        out_type=jax.ShapeDtypeStruct((b, d), jnp.float32),
        scratch_types=[
            pltpu.VMEM((b_per_w,), jnp.int32),
            pltpu.VMEM((b_per_w, d), jnp.float32),
            pltpu.SemaphoreType.DMA,
        ],
    )
    def gather_kernel(table_hbm, idx_hbm, out_hbm, idx_v, rows_v, sem):
        wid = lax.axis_index("s") * nc + lax.axis_index("c")
        base = wid * b_per_w
        pltpu.sync_copy(idx_hbm.at[pl.ds(base, b_per_w)], idx_v)
        pltpu.async_copy(table_hbm.at[idx_v], rows_v, sem).wait()
        pltpu.sync_copy(rows_v, out_hbm.at[pl.ds(base, b_per_w)])

    return gather_kernel(codebook_padded, inds)


def _decoder_body(lat_ref, q_ref, w0_ref, b0_ref, w1_ref, b1_ref, w2_ref,
                  b2_ref, st_ref, rec_ref, loss_ref):
    i = pl.program_id(0)

    @pl.when(i == 0)
    def _():
        loss_ref[...] = jnp.zeros((1, 1), jnp.float32)

    lat = lat_ref[...]
    q = q_ref[:, :lat.shape[1]]
    diff = q - lat
    st = lat + diff
    st_ref[...] = st
    loss_ref[...] += jnp.sum(diff * diff).reshape(1, 1)
    h = jnp.maximum(st @ w0_ref[...] + b0_ref[...], 0.0)
    h = jnp.maximum(h @ w1_ref[...] + b1_ref[...], 0.0)
    rec_ref[...] = jnp.tanh(h @ w2_ref[...] + b2_ref[...])


def _decode(latents, quantized, dec_w0, dec_b0, dec_w1, dec_b1, dec_w2,
            dec_b2):
    b, d = latents.shape
    h = dec_w0.shape[1]
    a = dec_w2.shape[1]
    nb = b // _BT
    full = lambda shape: pl.BlockSpec(shape, lambda i: (0,) * len(shape))
    return pl.pallas_call(
        _decoder_body,
        grid=(nb,),
        in_specs=[
            pl.BlockSpec((_BT, d), lambda i: (i, 0)),
            pl.BlockSpec((_BT, quantized.shape[1]), lambda i: (i, 0)),
            full((d, h)), full((1, h)),
            full((h, h)), full((1, h)),
            full((h, a)), full((1, a)),
        ],
        out_specs=[
            pl.BlockSpec((_BT, d), lambda i: (i, 0)),
            pl.BlockSpec((_BT, a), lambda i: (i, 0)),
            pl.BlockSpec((1, 1), lambda i: (0, 0)),
        ],
        out_shape=[
            jax.ShapeDtypeStruct((b, d), jnp.float32),
            jax.ShapeDtypeStruct((b, a), jnp.float32),
            jax.ShapeDtypeStruct((1, 1), jnp.float32),
        ],
    )(latents, quantized, dec_w0, dec_b0, dec_w1, dec_b1, dec_w2, dec_b2)


def kernel(action, enc_w0, enc_b0, enc_w1, enc_b1, enc_w2, enc_b2, codebook,
           dec_w0, dec_b0, dec_w1, dec_b1, dec_w2, dec_b2):
    b = action.shape[0]
    d = codebook.shape[1]
    beta = 0.25
    cbn = jnp.sum(codebook ** 2, axis=1)[None, :]
    latents, inds3 = _encode_argmin(
        action, enc_w0, enc_b0[None, :], enc_w1, enc_b1[None, :], enc_w2,
        enc_b2[None, :], codebook, cbn)
    inds = inds3.reshape(b)
    cb_padded = jnp.pad(codebook, ((0, 0), (0, 128 - d)))
    quantized = _sc_gather(cb_padded, inds)
    quantized_st, recons, loss_sum = _decode(
        latents, quantized, dec_w0, dec_b0[None, :], dec_w1, dec_b1[None, :],
        dec_w2, dec_b2[None, :])
    mse = loss_sum[0, 0] / (b * d)
    vq_loss = mse * beta + mse
    return (inds[:, None], quantized_st, recons, action, vq_loss)


# speedup vs baseline: 1.1064x; 1.1064x over previous
import functools

import jax
import jax.numpy as jnp
from jax import lax
from jax.experimental import pallas as pl
from jax.experimental.pallas import tpu as pltpu
from jax.experimental.pallas import tpu_sc as plsc

_BT = 512


def _encode_argmin_body(a_ref, w0_ref, b0_ref, w1_ref, b1_ref, w2_ref,
                        b2_ref, cb_ref, cbn_ref, lat_ref, ind_ref):
    h = jnp.maximum(a_ref[...] @ w0_ref[...] + b0_ref[...], 0.0)
    h = jnp.maximum(h @ w1_ref[...] + b1_ref[...], 0.0)
    lat = h @ w2_ref[...] + b2_ref[...]
    lat_ref[...] = lat
    fn = jnp.sum(lat ** 2, axis=1, keepdims=True)
    dots = lax.dot_general(lat, cb_ref[...], (((1,), (1,)), ((), ())))
    dist = (fn + cbn_ref[...]) - 2.0 * dots
    m = jnp.min(dist, axis=1, keepdims=True)
    k = dist.shape[1]
    iota = lax.broadcasted_iota(jnp.int32, dist.shape, 1)
    ind = jnp.min(jnp.where(dist == m, iota, k), axis=1)
    ind_ref[...] = ind.reshape(1, 1, _BT)


def _encode_argmin(action, enc_w0, enc_b0, enc_w1, enc_b1, enc_w2, enc_b2,
                   codebook, cbn):
    b, a = action.shape
    k, d = codebook.shape
    h = enc_w0.shape[1]
    nb = b // _BT
    full = lambda shape: pl.BlockSpec(shape, lambda i: (0,) * len(shape))
    return pl.pallas_call(
        _encode_argmin_body,
        grid=(nb,),
        in_specs=[
            pl.BlockSpec((_BT, a), lambda i: (i, 0)),
            full((a, h)), full((1, h)),
            full((h, h)), full((1, h)),
            full((h, d)), full((1, d)),
            full((k, d)), full((1, k)),
        ],
        out_specs=[
            pl.BlockSpec((_BT, d), lambda i: (i, 0)),
            pl.BlockSpec((1, 1, _BT), lambda i: (i, 0, 0)),
        ],
        out_shape=[
            jax.ShapeDtypeStruct((b, d), jnp.float32),
            jax.ShapeDtypeStruct((nb, 1, _BT), jnp.int32),
        ],
    )(action, enc_w0, enc_b0, enc_w1, enc_b1, enc_w2, enc_b2, codebook, cbn)


def _sc_gather(codebook_padded, inds):
    b = inds.shape[0]
    k, d = codebook_padded.shape
    nc, ns = 2, 16
    nw = nc * ns
    b_per_w = b // nw
    mesh = plsc.VectorSubcoreMesh(core_axis_name="c", subcore_axis_name="s")

    @functools.partial(
        pl.kernel, mesh=mesh,
        out_type=jax.ShapeDtypeStruct((b, d), jnp.float32),
        scratch_types=[
            pltpu.VMEM((b_per_w,), jnp.int32),
            pltpu.VMEM((b_per_w, d), jnp.float32),
            pltpu.SemaphoreType.DMA,
        ],
    )
    def gather_kernel(table_hbm, idx_hbm, out_hbm, idx_v, rows_v, sem):
        wid = lax.axis_index("s") * nc + lax.axis_index("c")
        base = wid * b_per_w
        pltpu.sync_copy(idx_hbm.at[pl.ds(base, b_per_w)], idx_v)
        pltpu.async_copy(table_hbm.at[idx_v], rows_v, sem).wait()
        pltpu.sync_copy(rows_v, out_hbm.at[pl.ds(base, b_per_w)])

    return gather_kernel(codebook_padded, inds)


def _decoder_body(lat_ref, q_ref, w0_ref, b0_ref, w1_ref, b1_ref, w2_ref,
                  b2_ref, st_ref, rec_ref, loss_ref):
    i = pl.program_id(0)

    @pl.when(i == 0)
    def _():
        loss_ref[...] = jnp.zeros((1, 1), jnp.float32)

    lat = lat_ref[...]
    q = q_ref[:, :lat.shape[1]]
    diff = q - lat
    st = lat + diff
    st_ref[...] = st
    loss_ref[...] += jnp.sum(diff * diff).reshape(1, 1)
    h = jnp.maximum(st @ w0_ref[...] + b0_ref[...], 0.0)
    h = jnp.maximum(h @ w1_ref[...] + b1_ref[...], 0.0)
    rec_ref[...] = jnp.tanh(h @ w2_ref[...] + b2_ref[...])


def _decode(latents, quantized, dec_w0, dec_b0, dec_w1, dec_b1, dec_w2,
            dec_b2):
    b, d = latents.shape
    h = dec_w0.shape[1]
    a = dec_w2.shape[1]
    nb = b // _BT
    full = lambda shape: pl.BlockSpec(shape, lambda i: (0,) * len(shape))
    return pl.pallas_call(
        _decoder_body,
        grid=(nb,),
        in_specs=[
            pl.BlockSpec((_BT, d), lambda i: (i, 0)),
            pl.BlockSpec((_BT, quantized.shape[1]), lambda i: (i, 0)),
            full((d, h)), full((1, h)),
            full((h, h)), full((1, h)),
            full((h, a)), full((1, a)),
        ],
        out_specs=[
            pl.BlockSpec((_BT, d), lambda i: (i, 0)),
            pl.BlockSpec((_BT, a), lambda i: (i, 0)),
            pl.BlockSpec((1, 1), lambda i: (0, 0)),
        ],
        out_shape=[
            jax.ShapeDtypeStruct((b, d), jnp.float32),
            jax.ShapeDtypeStruct((b, a), jnp.float32),
            jax.ShapeDtypeStruct((1, 1), jnp.float32),
        ],
    )(latents, quantized, dec_w0, dec_b0, dec_w1, dec_b1, dec_w2, dec_b2)


def kernel(action, enc_w0, enc_b0, enc_w1, enc_b1, enc_w2, enc_b2, codebook,
           dec_w0, dec_b0, dec_w1, dec_b1, dec_w2, dec_b2):
    b = action.shape[0]
    d = codebook.shape[1]
    beta = 0.25
    cbn = jnp.sum(codebook ** 2, axis=1)[None, :]
    latents, inds3 = _encode_argmin(
        action, enc_w0, enc_b0[None, :], enc_w1, enc_b1[None, :], enc_w2,
        enc_b2[None, :], codebook, cbn)
    inds = inds3.reshape(b)
    cb_padded = jnp.pad(codebook, ((0, 0), (0, 128 - d)))
    quantized = _sc_gather(cb_padded, inds)
    quantized_st, recons, loss_sum = _decode(
        latents, quantized, dec_w0, dec_b0[None, :], dec_w1, dec_b1[None, :],
        dec_w2, dec_b2[None, :])
    mse = loss_sum[0, 0] / (b * d)
    vq_loss = mse * beta + mse
    return (inds[:, None], quantized_st, recons, action, vq_loss)
